# baseline (speedup 1.0000x reference)
import jax
import jax.numpy as jnp
from jax.experimental import pallas as pl
from jax.experimental.pallas import tpu as pltpu

S, D = 2048, 768
H, DH = 12, 64
INNER = H * DH
E = 8
HID = 1024
EPS = 1e-5

BS1 = 256
BQ = 256
BLK = 256
NB = S // BLK


def _ln(x, scale, bias):
    m = jnp.mean(x, axis=-1, keepdims=True)
    v = jnp.mean((x - m) ** 2, axis=-1, keepdims=True)
    return (x - m) * jax.lax.rsqrt(v + EPS) * scale + bias


def _k1(x_ref, s_ref, b_ref, w_ref, out_ref):
    h = _ln(x_ref[...], s_ref[...], b_ref[...])
    out_ref[...] = jnp.dot(h, w_ref[...], preferred_element_type=jnp.float32)


def _qkv(x, ln1_scale, ln1_bias, wqkv):
    return pl.pallas_call(
        _k1,
        grid=(S // BS1, 3),
        in_specs=[
            pl.BlockSpec((BS1, D), lambda i, j: (i, 0)),
            pl.BlockSpec((1, D), lambda i, j: (0, 0)),
            pl.BlockSpec((1, D), lambda i, j: (0, 0)),
            pl.BlockSpec((D, INNER), lambda i, j: (0, j)),
        ],
        out_specs=pl.BlockSpec((BS1, INNER), lambda i, j: (i, j)),
        out_shape=jax.ShapeDtypeStruct((S, 3 * INNER), jnp.float32),
    )(x, ln1_scale.reshape(1, D), ln1_bias.reshape(1, D), wqkv)


def _k2(q_ref, k_ref, v_ref, o_ref):
    q = q_ref[0]
    k = k_ref[0]
    s = jax.lax.dot_general(q, k, (((1,), (1,)), ((), ())),
                            preferred_element_type=jnp.float32) * (DH ** -0.5)
    m = jnp.max(s, axis=-1, keepdims=True)
    p = jnp.exp(s - m)
    p = p / jnp.sum(p, axis=-1, keepdims=True)
    o_ref[0] = jnp.dot(p, v_ref[0], preferred_element_type=jnp.float32)


def _attn(q, k, v):
    return pl.pallas_call(
        _k2,
        grid=(H, S // BQ),
        in_specs=[
            pl.BlockSpec((1, BQ, DH), lambda h, i: (h, i, 0)),
            pl.BlockSpec((1, S, DH), lambda h, i: (h, 0, 0)),
            pl.BlockSpec((1, S, DH), lambda h, i: (h, 0, 0)),
        ],
        out_specs=pl.BlockSpec((1, BQ, DH), lambda h, i: (h, i, 0)),
        out_shape=jax.ShapeDtypeStruct((H, S, DH), jnp.float32),
    )(q, k, v)


def _k3(x_ref, o_ref, wo_ref, bo_ref, s_ref, b_ref, wg_ref,
        x1_ref, h2_ref, rw_ref, we_ref):
    x1 = x_ref[...] + jnp.dot(o_ref[...], wo_ref[...],
                              preferred_element_type=jnp.float32) + bo_ref[...]
    x1_ref[...] = x1
    h2 = _ln(x1, s_ref[...], b_ref[...])
    h2_ref[...] = h2
    g = jnp.dot(h2, wg_ref[...], preferred_element_type=jnp.float32)
    m = jnp.max(g, axis=-1, keepdims=True)
    p = jnp.exp(g - m)
    p = p / jnp.sum(p, axis=-1, keepdims=True)
    rw_ref[...] = p
    lane = jax.lax.broadcasted_iota(jnp.int32, p.shape, 1)
    m1 = jnp.max(p, axis=-1, keepdims=True)
    i1 = jnp.min(jnp.where(p == m1, lane, E), axis=-1, keepdims=True)
    mask1 = lane == i1
    p2 = jnp.where(mask1, -1.0, p)
    m2 = jnp.max(p2, axis=-1, keepdims=True)
    i2 = jnp.min(jnp.where(p2 == m2, lane, E), axis=-1, keepdims=True)
    mask2 = lane == i2
    tot = m1 + m2
    we_ref[...] = jnp.where(mask1, m1 / tot, jnp.where(mask2, m2 / tot, 0.0))


def _proj_gate(x, o, Wo, bo, ln2_scale, ln2_bias, Wg):
    return pl.pallas_call(
        _k3,
        grid=(S // BS1,),
        in_specs=[
            pl.BlockSpec((BS1, D), lambda i: (i, 0)),
            pl.BlockSpec((BS1, INNER), lambda i: (i, 0)),
            pl.BlockSpec((INNER, D), lambda i: (0, 0)),
            pl.BlockSpec((1, D), lambda i: (0, 0)),
            pl.BlockSpec((1, D), lambda i: (0, 0)),
            pl.BlockSpec((1, D), lambda i: (0, 0)),
            pl.BlockSpec((D, E), lambda i: (0, 0)),
        ],
        out_specs=[
            pl.BlockSpec((BS1, D), lambda i: (i, 0)),
            pl.BlockSpec((BS1, D), lambda i: (i, 0)),
            pl.BlockSpec((BS1, E), lambda i: (i, 0)),
            pl.BlockSpec((BS1, E), lambda i: (i, 0)),
        ],
        out_shape=[
            jax.ShapeDtypeStruct((S, D), jnp.float32),
            jax.ShapeDtypeStruct((S, D), jnp.float32),
            jax.ShapeDtypeStruct((S, E), jnp.float32),
            jax.ShapeDtypeStruct((S, E), jnp.float32),
        ],
    )(x, o, Wo, bo.reshape(1, D), ln2_scale.reshape(1, D),
      ln2_bias.reshape(1, D), Wg)


def _k4d(h2_ref, x1_ref, we_ref, np_ref, w1_ref, w3_ref, w2_ref,
         fin_ref, bs_ref):
    e = pl.program_id(1)
    xb = h2_ref[...]
    h1 = jnp.dot(xb, w1_ref[0], preferred_element_type=jnp.float32)
    h1 = h1 * (1.0 / (1.0 + jnp.exp(-h1)))
    h3 = jnp.dot(xb, w3_ref[0], preferred_element_type=jnp.float32)
    y = jnp.dot(h1 * h3, w2_ref[0], preferred_element_type=jnp.float32)
    lane = jax.lax.broadcasted_iota(jnp.int32, (BLK, E), 1)
    w = jnp.sum(jnp.where(lane == e, we_ref[...], 0.0), axis=1, keepdims=True)
    contrib = y * w

    @pl.when(e == 0)
    def _():
        fin_ref[...] = x1_ref[...] + contrib

    @pl.when(e != 0)
    def _():
        fin_ref[...] = fin_ref[...] + contrib

    row = jnp.sum(y * (w * np_ref[...]), axis=0, keepdims=True)
    rmask = jax.lax.broadcasted_iota(jnp.int32, (8, D), 0) == 0
    bs_ref[...] = jnp.where(rmask, row, 0.0)


def _moe_dense(h2, x1, we, nonpad):
    def wspec(shape):
        return pl.BlockSpec((1,) + shape, lambda b, e: (e, 0, 0))

    return pl.pallas_call(
        _k4d,
        grid=(NB, E),
        in_specs=[
            pl.BlockSpec((BLK, D), lambda b, e: (b, 0)),
            pl.BlockSpec((BLK, D), lambda b, e: (b, 0)),
            pl.BlockSpec((BLK, E), lambda b, e: (b, 0)),
            pl.BlockSpec((BLK, 1), lambda b, e: (b, 0)),
            wspec((D, HID)),
            wspec((D, HID)),
            wspec((HID, D)),
        ],
        out_specs=[
            pl.BlockSpec((BLK, D), lambda b, e: (b, 0)),
            pl.BlockSpec((8, D), lambda b, e: (b * E + e, 0)),
        ],
        out_shape=[
            jax.ShapeDtypeStruct((S, D), jnp.float32),
            jax.ShapeDtypeStruct((NB * E * 8, D), jnp.float32),
        ],
    )


def _k5(mc_ref, bs_ref, wc_ref, bc_ref, out_ref):
    embs = jnp.dot(mc_ref[...], bs_ref[...], preferred_element_type=jnp.float32)
    out_ref[...] = jnp.dot(embs, wc_ref[...],
                           preferred_element_type=jnp.float32) + bc_ref[...]


def _logits(mcum, bsums, Wc, bc):
    R = bsums.shape[0]
    return pl.pallas_call(
        _k5,
        grid=(1,),
        in_specs=[
            pl.BlockSpec((E, R), lambda i: (0, 0)),
            pl.BlockSpec((R, D), lambda i: (0, 0)),
            pl.BlockSpec((D, 1), lambda i: (0, 0)),
            pl.BlockSpec((1, 1), lambda i: (0, 0)),
        ],
        out_specs=pl.BlockSpec((E, 1), lambda i: (0, 0)),
        out_shape=jax.ShapeDtypeStruct((E, 1), jnp.float32),
    )(mcum, bsums, Wc, bc.reshape(1, 1))


def kernel(x, tgt_pad, tgt_mask_id_bool, ln1_scale, ln1_bias, ln2_scale,
           ln2_bias, Wq, Wk, Wv, Wo, bo, Wg, w1, w2, w3, Wc, bc):
    x2 = x.reshape(S, D)
    wqkv = jnp.concatenate([Wq, Wk, Wv], axis=1)
    qkv = _qkv(x2, ln1_scale, ln1_bias, wqkv)
    q = qkv[:, :INNER].reshape(S, H, DH).transpose(1, 0, 2)
    k = qkv[:, INNER:2 * INNER].reshape(S, H, DH).transpose(1, 0, 2)
    v = qkv[:, 2 * INNER:].reshape(S, H, DH).transpose(1, 0, 2)
    o = _attn(q, k, v).transpose(1, 0, 2).reshape(S, INNER)
    x1, h2, rw, we = _proj_gate(x2, o, Wo, bo, ln2_scale, ln2_bias, Wg)

    nonpad = (~(tgt_pad | tgt_mask_id_bool)).astype(jnp.float32).reshape(S, 1)
    denom = jnp.maximum(jnp.sum(nonpad), 1.0)

    x_out, bsums = _moe_dense(h2, x1, we, nonpad)(
        h2, x1, we, nonpad, w1, w3, w2)

    ep = jnp.arange(E)
    rows = jnp.repeat(jnp.tile(jnp.arange(E), NB), 8)
    first = (jnp.tile(jnp.arange(8), NB * E) == 0)
    mcum = ((rows[None, :] <= ep[:, None]) & first[None, :]).astype(jnp.float32) / denom
    logits = _logits(mcum, bsums, Wc, bc)

    return (x_out.reshape(1, S, D), logits.reshape(E, 1, 1),
            rw.reshape(1, S, E))
